# baseline (speedup 1.0000x reference)
import jax
import jax.numpy as jnp
from jax.experimental import pallas as pl
from jax.experimental.pallas import tpu as pltpu

_B, _C, _D, _H, _W = 16, 4, 96, 96, 96
_BC = _B * _C
_SP = _D * _H * _W
_LANES = 128
_ROWS = _SP // _LANES


def _loss_body(inp_ref, lab_ref, out_ref):
    x = inp_ref[0]
    lab = lab_ref[0]
    d = x - lab
    ssq = jnp.sum(d * d, axis=0, keepdims=True)
    sab = jnp.sum(jnp.abs(lab), axis=0, keepdims=True)
    ssq_s = jnp.sum(ssq, axis=1, keepdims=True)
    sab_s = jnp.sum(sab, axis=1, keepdims=True)
    ratio = ssq_s / sab_s
    out_ref[0] = jnp.broadcast_to(ratio, (1, _LANES))


def kernel(input, label):
    inp3 = input.reshape(_BC, _ROWS, _LANES)
    lab3 = label.reshape(_BC, _ROWS, _LANES)
    ratios = pl.pallas_call(
        _loss_body,
        out_shape=jax.ShapeDtypeStruct((_BC, 1, _LANES), jnp.float32),
        grid=(_BC,),
        in_specs=[
            pl.BlockSpec((1, _ROWS, _LANES), lambda i: (i, 0, 0)),
            pl.BlockSpec((1, _ROWS, _LANES), lambda i: (i, 0, 0)),
        ],
        out_specs=pl.BlockSpec((1, 1, _LANES), lambda i: (i, 0, 0)),
        compiler_params=pltpu.CompilerParams(
            dimension_semantics=("parallel",),
        ),
        name="my_loss",
    )(inp3, lab3)
    loss = jnp.sum(ratios[:, 0, 0]) * (1.0 / (_B * _C))
    return loss.reshape(1)


# speedup vs baseline: 1.0175x; 1.0175x over previous
import jax
import jax.numpy as jnp
from jax.experimental import pallas as pl
from jax.experimental.pallas import tpu as pltpu

_B, _C, _D, _H, _W = 16, 4, 96, 96, 96
_BC = _B * _C
_SP = _D * _H * _W
_LANES = 128
_ROWS = _SP // _LANES


_NS = 4
_RC = _ROWS // _NS


def _loss_body(*refs):
    in_refs = refs[:_NS]
    lab_refs = refs[_NS:2 * _NS]
    out_ref = refs[2 * _NS]
    ssq = jnp.zeros((1, _LANES), jnp.float32)
    sab = jnp.zeros((1, _LANES), jnp.float32)
    for x_ref, l_ref in zip(in_refs, lab_refs):
        x = x_ref[0, 0]
        lab = l_ref[0, 0]
        d = x - lab
        ssq = ssq + jnp.sum(d * d, axis=0, keepdims=True)
        sab = sab + jnp.sum(jnp.abs(lab), axis=0, keepdims=True)
    ssq_s = jnp.sum(ssq, axis=1, keepdims=True)
    sab_s = jnp.sum(sab, axis=1, keepdims=True)
    ratio = ssq_s / sab_s
    out_ref[0] = jnp.broadcast_to(ratio, (1, _LANES))


def kernel(input, label):
    inp4 = input.reshape(_BC, _NS, _RC, _LANES)
    lab4 = label.reshape(_BC, _NS, _RC, _LANES)
    specs = [
        pl.BlockSpec((1, 1, _RC, _LANES), lambda i, k=k: (i, k, 0, 0))
        for k in range(_NS)
    ]
    ratios = pl.pallas_call(
        _loss_body,
        out_shape=jax.ShapeDtypeStruct((_BC, 1, _LANES), jnp.float32),
        grid=(_BC,),
        in_specs=specs + specs,
        out_specs=pl.BlockSpec((1, 1, _LANES), lambda i: (i, 0, 0)),
        compiler_params=pltpu.CompilerParams(
            dimension_semantics=("parallel",),
        ),
        name="my_loss",
    )(*([inp4] * _NS + [lab4] * _NS))
    loss = jnp.sum(ratios[:, 0, 0]) * (1.0 / (_B * _C))
    return loss.reshape(1)


# speedup vs baseline: 4.2882x; 4.2144x over previous
import jax
import jax.numpy as jnp
from jax.experimental import pallas as pl
from jax.experimental.pallas import tpu as pltpu

_B, _C, _D, _H, _W = 16, 4, 96, 96, 96
_BC = _B * _C
_ROWS = _D * _H


def _loss_body(inp_ref, lab_ref, out_ref):
    x = inp_ref[0]
    lab = lab_ref[0]
    d = x - lab
    ssq = jnp.sum(d * d, axis=0, keepdims=True)
    sab = jnp.sum(jnp.abs(lab), axis=0, keepdims=True)
    ssq_s = jnp.sum(ssq, axis=1, keepdims=True)
    sab_s = jnp.sum(sab, axis=1, keepdims=True)
    ratio = ssq_s / sab_s
    out_ref[0] = jnp.broadcast_to(ratio, (1, _W))


def kernel(input, label):
    inp3 = input.reshape(_BC, _ROWS, _W)
    lab3 = label.reshape(_BC, _ROWS, _W)
    ratios = pl.pallas_call(
        _loss_body,
        out_shape=jax.ShapeDtypeStruct((_BC, 1, _W), jnp.float32),
        grid=(_BC,),
        in_specs=[
            pl.BlockSpec((1, _ROWS, _W), lambda i: (i, 0, 0)),
            pl.BlockSpec((1, _ROWS, _W), lambda i: (i, 0, 0)),
        ],
        out_specs=pl.BlockSpec((1, 1, _W), lambda i: (i, 0, 0)),
        compiler_params=pltpu.CompilerParams(
            dimension_semantics=("parallel",),
        ),
        name="my_loss",
    )(inp3, lab3)
    loss = jnp.sum(ratios[:, 0, 0]) * (1.0 / (_B * _C))
    return loss.reshape(1)


# speedup vs baseline: 4.4305x; 1.0332x over previous
import jax
import jax.numpy as jnp
from jax.experimental import pallas as pl
from jax.experimental.pallas import tpu as pltpu

_B, _C, _D, _H, _W = 16, 4, 96, 96, 96
_BC = _B * _C
_ROWS = _D * _H


_CPB = 2


def _loss_body(inp_ref, lab_ref, out_ref):
    x = inp_ref[...]
    lab = lab_ref[...]
    d = x - lab
    ssq = jnp.sum(d * d, axis=1)
    sab = jnp.sum(jnp.abs(lab), axis=1)
    ssq_s = jnp.sum(ssq, axis=1, keepdims=True)
    sab_s = jnp.sum(sab, axis=1, keepdims=True)
    ratio = ssq_s / sab_s
    out_ref[...] = jnp.broadcast_to(ratio[:, :, None], (_CPB, 1, _W))


def kernel(input, label):
    inp3 = input.reshape(_BC, _ROWS, _W)
    lab3 = label.reshape(_BC, _ROWS, _W)
    ratios = pl.pallas_call(
        _loss_body,
        out_shape=jax.ShapeDtypeStruct((_BC, 1, _W), jnp.float32),
        grid=(_BC // _CPB,),
        in_specs=[
            pl.BlockSpec((_CPB, _ROWS, _W), lambda i: (i, 0, 0)),
            pl.BlockSpec((_CPB, _ROWS, _W), lambda i: (i, 0, 0)),
        ],
        out_specs=pl.BlockSpec((_CPB, 1, _W), lambda i: (i, 0, 0)),
        compiler_params=pltpu.CompilerParams(
            dimension_semantics=("parallel",),
            vmem_limit_bytes=48 * 1024 * 1024,
        ),
        name="my_loss",
    )(inp3, lab3)
    loss = jnp.sum(ratios[:, 0, 0]) * (1.0 / (_B * _C))
    return loss.reshape(1)


# speedup vs baseline: 4.4347x; 1.0009x over previous
import jax
import jax.numpy as jnp
from jax.experimental import pallas as pl
from jax.experimental.pallas import tpu as pltpu

_B, _C, _D, _H, _W = 16, 4, 96, 96, 96
_BC = _B * _C
_ROWS = _D * _H
_CPB = 2
_NJ = _BC // 2 // _CPB


def _loss_body(inp_ref, lab_ref, out_ref, acc_ref):
    j = pl.program_id(1)

    @pl.when(j == 0)
    def _():
        acc_ref[...] = jnp.zeros_like(acc_ref)

    x = inp_ref[...]
    lab = lab_ref[...]
    d = x - lab
    ssq = jnp.sum(d * d, axis=1)
    sab = jnp.sum(jnp.abs(lab), axis=1)
    ssq_s = jnp.sum(ssq, axis=1, keepdims=True)
    sab_s = jnp.sum(sab, axis=1, keepdims=True)
    ratio = ssq_s / sab_s
    step_sum = jnp.sum(ratio, axis=0, keepdims=True)
    acc_ref[...] += jnp.broadcast_to(step_sum, (1, _W))

    @pl.when(j == _NJ - 1)
    def _():
        out_ref[...] = acc_ref[...].reshape(1, 1, _W) * (1.0 / (_B * _C))


def kernel(input, label):
    inp3 = input.reshape(_BC, _ROWS, _W)
    lab3 = label.reshape(_BC, _ROWS, _W)
    partials = pl.pallas_call(
        _loss_body,
        out_shape=jax.ShapeDtypeStruct((2, 1, _W), jnp.float32),
        grid=(2, _NJ),
        in_specs=[
            pl.BlockSpec((_CPB, _ROWS, _W), lambda c, j: (c * _NJ + j, 0, 0)),
            pl.BlockSpec((_CPB, _ROWS, _W), lambda c, j: (c * _NJ + j, 0, 0)),
        ],
        out_specs=pl.BlockSpec((1, 1, _W), lambda c, j: (c, 0, 0)),
        scratch_shapes=[pltpu.VMEM((1, _W), jnp.float32)],
        compiler_params=pltpu.CompilerParams(
            dimension_semantics=("parallel", "arbitrary"),
            vmem_limit_bytes=48 * 1024 * 1024,
        ),
        name="my_loss",
    )(inp3, lab3)
    loss = partials[0, 0, 0] + partials[1, 0, 0]
    return loss.reshape(1)
